# baseline (speedup 1.0000x reference)
import functools

import jax
import jax.numpy as jnp
from jax import lax
from jax.experimental import pallas as pl
from jax.experimental.pallas import tpu as pltpu
from jax.experimental.pallas import tpu_sc as plsc

_INFO = plsc.get_sparse_core_info()
_NC, _NS = _INFO.num_cores, _INFO.num_subcores
_NW = _NC * _NS

_CHUNK = 64


@functools.lru_cache(maxsize=None)
def _make_gather(n_rows: int, d: int, n_samples: int):
    assert n_samples % (_NW * _CHUNK) == 0
    b_per_w = n_samples // _NW
    n_chunks = b_per_w // _CHUNK
    mesh = plsc.VectorSubcoreMesh(core_axis_name="c", subcore_axis_name="s")

    @functools.partial(
        pl.kernel,
        mesh=mesh,
        out_type=jax.ShapeDtypeStruct((n_samples, d), jnp.float32),
        scratch_types=[
            pltpu.VMEM((n_chunks, _CHUNK), jnp.int32),
            pltpu.VMEM((2, _CHUNK, d), jnp.float32),
            pltpu.SemaphoreType.DMA,
            pltpu.SemaphoreType.DMA,
            pltpu.SemaphoreType.DMA,
            pltpu.SemaphoreType.DMA,
        ],
        compiler_params=pltpu.CompilerParams(use_tc_tiling_on_sc=False),
    )
    def gather(table_hbm, idx_hbm, out_hbm, idx_v, rows_v, g0, g1, s0, s1):
        wid = lax.axis_index("s") * _NC + lax.axis_index("c")
        base = wid * b_per_w
        gsems = (g0, g1)
        ssems = (s0, s1)
        pltpu.sync_copy(idx_hbm.at[pl.ds(wid * n_chunks, n_chunks)], idx_v)

        def start_gather(c):
            return pltpu.async_copy(
                table_hbm.at[idx_v.at[c]], rows_v.at[c % 2], gsems[c % 2]
            )

        def start_store(c):
            return pltpu.async_copy(
                rows_v.at[c % 2],
                out_hbm.at[pl.ds(base + c * _CHUNK, _CHUNK)],
                ssems[c % 2],
            )

        gathers = [None] * n_chunks
        stores = [None] * n_chunks
        gathers[0] = start_gather(0)
        for c in range(n_chunks):
            gathers[c].wait()
            if c >= 1:
                stores[c - 1].wait()
            if c + 1 < n_chunks:
                gathers[c + 1] = start_gather(c + 1)
            stores[c] = start_store(c)
        stores[n_chunks - 1].wait()

    return gather


@jax.jit
def kernel(images, indices):
    n, c, h, w = images.shape
    d = c * h * w
    n_samples = indices.shape[0]
    table = images.reshape(n, d)
    idx = indices.astype(jnp.int32).reshape(n_samples // _CHUNK, _CHUNK)
    out = _make_gather(n, d, n_samples)(table, idx)
    return out.reshape(n_samples, c, h, w)


# speedup vs baseline: 1.6453x; 1.6453x over previous
import functools

import jax
import jax.numpy as jnp
from jax import lax
from jax.experimental import pallas as pl
from jax.experimental.pallas import tpu as pltpu
from jax.experimental.pallas import tpu_sc as plsc

_INFO = plsc.get_sparse_core_info()
_NC, _NS, _NL = _INFO.num_cores, _INFO.num_subcores, _INFO.num_lanes
_NW = _NC * _NS

_UNROLL = 8


@functools.lru_cache(maxsize=None)
def _make_gather(d: int, n_rows: int, n_samples: int):
    assert n_samples % (_NL * _UNROLL) == 0
    r_per_w = -(-d // _NW)
    mesh = plsc.VectorSubcoreMesh(core_axis_name="c", subcore_axis_name="s")

    @functools.partial(
        pl.kernel,
        mesh=mesh,
        out_type=jax.ShapeDtypeStruct((d, n_samples), jnp.float32),
        scratch_types=[
            pltpu.VMEM((n_samples,), jnp.int32),
            pltpu.VMEM((n_rows,), jnp.float32),
            pltpu.VMEM((n_samples,), jnp.float32),
        ],
        compiler_params=pltpu.CompilerParams(use_tc_tiling_on_sc=False, needs_layout_passes=False),
    )
    def gather(table_hbm, idx_hbm, out_hbm, idx_v, row_v, out_v):
        wid = lax.axis_index("s") * _NC + lax.axis_index("c")
        pltpu.sync_copy(idx_hbm, idx_v)

        def do_row(p):
            pltpu.sync_copy(table_hbm.at[p], row_v)

            def step(i, carry):
                base = i * (_NL * _UNROLL)
                for u in range(_UNROLL):
                    off = base + u * _NL
                    idx16 = idx_v[pl.ds(off, _NL)]
                    out_v[pl.ds(off, _NL)] = plsc.load_gather(row_v, [idx16])
                return carry

            lax.fori_loop(0, n_samples // (_NL * _UNROLL), step, 0)
            pltpu.sync_copy(out_v, out_hbm.at[p])

        for r in range(r_per_w):
            p = wid + r * _NW
            if (r + 1) * _NW <= d:
                do_row(p)
            else:
                @pl.when(p < d)
                def _():
                    do_row(p)

    return gather


@jax.jit
def kernel(images, indices):
    n, c, h, w = images.shape
    d = c * h * w
    n_samples = indices.shape[0]
    table_t = images.reshape(n, d).T
    idx = indices.astype(jnp.int32)
    out_t = _make_gather(d, n, n_samples)(table_t, idx)
    return out_t.reshape(h, w, c, n_samples).transpose(3, 2, 0, 1)
